# baseline (speedup 1.0000x reference)
import functools

import jax
import jax.numpy as jnp
from jax import lax
from jax.experimental import pallas as pl
from jax.experimental.pallas import tpu as pltpu
from jax.experimental.pallas import tpu_sc as plsc

N = 10000
E = 160000
H = 256
HALF = 128
ROWS = 1000
TILES = N // ROWS
NSUB = 16
RPS = 640
NPAD = NSUB * RPS

@functools.cache
def _sc_mesh():
    return plsc.VectorSubcoreMesh(core_axis_name="c", subcore_axis_name="s")


_CHUNK = 80


def _make_sc_body(nchunks, tail, feature_split, eps_sub):
    def body(tbl_hbm, src_hbm, dst_hbm, out_hbm, srcv3, dstv3, gv3, rows3,
             srcT, dstT, zv, acc, sem_i, sem_g0, sem_g1, sem_g2,
             sem_s0, sem_s1, sem_s2):
        c = lax.axis_index("c")
        s = lax.axis_index("s")

        @pl.loop(0, 64)
        def _(i):
            @pl.loop(0, HALF, step=16)
            def _(j):
                zv[i, pl.ds(j, 16)] = jnp.zeros((16,), jnp.float32)

        base_r = s * RPS

        @pl.loop(0, RPS // 64)
        def _(k):
            pltpu.sync_copy(zv, acc.at[pl.ds(base_r + k * 64, 64)])

        plsc.subcore_barrier()

        if feature_split:
            base_e = s * eps_sub
        else:
            base_e = (c * NSUB + s) * eps_sub

        gsems = (sem_g0, sem_g1, sem_g2)
        ssems = (sem_s0, sem_s1, sem_s2)

        def fetch_idx(k, slot):
            off = base_e + k * _CHUNK
            a = pltpu.async_copy(src_hbm.at[pl.ds(off, _CHUNK)],
                                 srcv3.at[slot], sem_i)
            b = pltpu.async_copy(dst_hbm.at[pl.ds(off, _CHUNK)],
                                 dstv3.at[slot], sem_i)
            a.wait()
            b.wait()
            if feature_split:
                @pl.loop(0, _CHUNK, step=16)
                def _(j):
                    gv3[slot, pl.ds(j, 16)] = srcv3[slot, pl.ds(j, 16)] * 2 + c

        def gidx(slot):
            return gv3.at[slot] if feature_split else srcv3.at[slot]

        def fire_gather(slot):
            pltpu.async_copy(tbl_hbm.at[gidx(slot)], rows3.at[slot],
                             gsems[slot])

        def wait_gather(slot):
            pltpu.make_async_copy(tbl_hbm.at[gidx(slot)], rows3.at[slot],
                                  gsems[slot]).wait()

        def fire_scatter(slot):
            pltpu.async_copy(rows3.at[slot], acc.at[dstv3.at[slot]],
                             ssems[slot], add=True)

        def wait_scatter(slot):
            pltpu.make_async_copy(rows3.at[slot], acc.at[dstv3.at[slot]],
                                  ssems[slot]).wait()

        def step(kcur, s_cur, s_nxt, s_prev):
            @pl.when(kcur + 1 < nchunks)
            def _():
                fetch_idx(kcur + 1, s_nxt)

            @pl.when(kcur < nchunks)
            def _():
                wait_gather(s_cur)

            @pl.when(kcur + 1 < nchunks)
            def _():
                fire_gather(s_nxt)

            @pl.when(jnp.logical_and(kcur >= 1, kcur - 1 < nchunks))
            def _():
                wait_scatter(s_prev)

            @pl.when(kcur < nchunks)
            def _():
                fire_scatter(s_cur)

        fetch_idx(0, 0)
        fire_gather(0)

        @pl.loop(0, (nchunks + 3) // 3)
        def _(i):
            step(3 * i, 0, 1, 2)
            step(3 * i + 1, 1, 2, 0)
            step(3 * i + 2, 2, 0, 1)

        if tail:
            off = base_e + nchunks * _CHUNK
            a = pltpu.async_copy(src_hbm.at[pl.ds(off, tail)], srcT, sem_i)
            b = pltpu.async_copy(dst_hbm.at[pl.ds(off, tail)], dstT, sem_i)
            a.wait()
            b.wait()
            rt = rows3.at[0, pl.ds(0, tail)]
            pltpu.async_copy(tbl_hbm.at[srcT], rt, sem_g0).wait()
            pltpu.sync_copy(rt, acc.at[dstT], add=True)

        plsc.subcore_barrier()
        pltpu.sync_copy(acc.at[pl.ds(base_r, RPS)],
                        out_hbm.at[c, pl.ds(base_r, RPS)])

    return body


def _sc_call(body, tail, tbl, src, dst):
    kern = pl.kernel(
        body,
        out_type=jax.ShapeDtypeStruct((2, NPAD, HALF), jnp.float32),
        mesh=_sc_mesh(),
        scratch_types=[
            pltpu.VMEM((3, _CHUNK), jnp.int32),
            pltpu.VMEM((3, _CHUNK), jnp.int32),
            pltpu.VMEM((3, _CHUNK), jnp.int32),
            pltpu.VMEM((3, _CHUNK, HALF), jnp.float32),
            pltpu.VMEM((max(tail, 8),), jnp.int32),
            pltpu.VMEM((max(tail, 8),), jnp.int32),
            pltpu.VMEM((64, HALF), jnp.float32),
            pltpu.VMEM_SHARED((NPAD, HALF), jnp.float32),
            pltpu.SemaphoreType.DMA,
            pltpu.SemaphoreType.DMA,
            pltpu.SemaphoreType.DMA,
            pltpu.SemaphoreType.DMA,
            pltpu.SemaphoreType.DMA,
            pltpu.SemaphoreType.DMA,
            pltpu.SemaphoreType.DMA,
        ],
    )
    return kern(tbl, src, dst)


_EPS_SUB = E // NSUB
_EPS_SUB0 = E // (2 * NSUB)
_MAIN_BODY = _make_sc_body(_EPS_SUB // _CHUNK, _EPS_SUB % _CHUNK, True,
                           _EPS_SUB)
_BLK0_BODY = _make_sc_body(_EPS_SUB0 // _CHUNK, _EPS_SUB0 % _CHUNK, False,
                           _EPS_SUB0)


def _sc_segsum(h2, src, dst):
    return _sc_call(_MAIN_BODY, _EPS_SUB % _CHUNK, h2, src, dst)


def _sc_segsum0(x128, src, dst):
    return _sc_call(_BLK0_BODY, _EPS_SUB0 % _CHUNK, x128, src, dst)


_RSN = 1.0 / float(N) ** 0.5


def _bn_apply(v, mrow, vrow, g, b):
    return (v - mrow) * lax.rsqrt(vrow + 1e-5) * g + b


def _leaky(v):
    return jnp.where(v >= 0, v, 0.01 * v)


def _tc_block_body(h_ref, nb_ref, epsb, w1, b1, w2, b2, g1, be1, g2, be2,
                   out_ref, zbuf, stats):
    p = pl.program_id(0)
    t = pl.program_id(1)
    nf = jnp.float32(N)

    @pl.when(jnp.logical_and(p == 0, t == 0))
    def _():
        stats[...] = jnp.zeros_like(stats)

    @pl.when(p == 0)
    def _():
        nb = jnp.concatenate([nb_ref[0], nb_ref[1]], axis=1)
        agg = h_ref[...] * epsb[...] + nb
        z = jnp.maximum(jnp.dot(agg, w1[...], preferred_element_type=jnp.float32,
                            precision=lax.Precision.HIGHEST)
                        + b1[...], 0.0)
        z = jnp.maximum(jnp.dot(z, w2[...], preferred_element_type=jnp.float32,
                            precision=lax.Precision.HIGHEST)
                        + b2[...], 0.0)
        zbuf[pl.ds(t * ROWS, ROWS), :] = z
        stats[0:1, :] += jnp.sum(z, axis=0, keepdims=True)
        stats[1:2, :] += jnp.sum(z * z, axis=0, keepdims=True)

    @pl.when(p == 1)
    def _():
        z = zbuf[pl.ds(t * ROWS, ROWS), :]
        m1 = stats[0:1, :] / nf
        v1 = stats[1:2, :] / nf - m1 * m1
        w = _leaky(_bn_apply(z, m1, v1, g1[...], be1[...])) * _RSN
        zbuf[pl.ds(t * ROWS, ROWS), :] = w
        stats[2:3, :] += jnp.sum(w, axis=0, keepdims=True)
        stats[3:4, :] += jnp.sum(w * w, axis=0, keepdims=True)

    @pl.when(p == 2)
    def _():
        w = zbuf[pl.ds(t * ROWS, ROWS), :]
        m2 = stats[2:3, :] / nf
        v2 = stats[3:4, :] / nf - m2 * m2
        out_ref[...] = _bn_apply(w, m2, v2, g2[...], be2[...])


def _tc_block(h, nb2, epsb, w1, b1, w2, b2, g1, be1, g2, be2):
    row_spec = pl.BlockSpec((ROWS, H), lambda p, t: (t, 0))
    vec_spec = pl.BlockSpec((1, H), lambda p, t: (0, 0))
    mat_spec = pl.BlockSpec((H, H), lambda p, t: (0, 0))
    return pl.pallas_call(
        _tc_block_body,
        grid=(3, TILES),
        in_specs=[
            row_spec,
            pl.BlockSpec((2, ROWS, HALF), lambda p, t: (0, t, 0)),
            vec_spec, mat_spec, vec_spec, mat_spec, vec_spec,
            vec_spec, vec_spec, vec_spec, vec_spec,
        ],
        out_specs=row_spec,
        out_shape=jax.ShapeDtypeStruct((N, H), jnp.float32),
        scratch_shapes=[
            pltpu.VMEM((N, H), jnp.float32),
            pltpu.VMEM((8, H), jnp.float32),
        ],
    )(h, nb2, epsb, w1, b1, w2, b2, g1, be1, g2, be2)


def _tc_block0_body(x_ref, s2_ref, epsb, w1, b1, w2, b2, g1, be1, g2, be2,
                    out_ref, zbuf, stats):
    p = pl.program_id(0)
    t = pl.program_id(1)
    nf = jnp.float32(N)

    @pl.when(jnp.logical_and(p == 0, t == 0))
    def _():
        stats[...] = jnp.zeros_like(stats)

    @pl.when(p == 0)
    def _():
        s = (s2_ref[0] + s2_ref[1])[:, 0:1]
        agg = x_ref[...] * epsb[...] + s
        z = jnp.maximum(agg * w1[...] + b1[...], 0.0)
        z = jnp.maximum(jnp.dot(z, w2[...], preferred_element_type=jnp.float32,
                            precision=lax.Precision.HIGHEST)
                        + b2[...], 0.0)
        zbuf[pl.ds(t * ROWS, ROWS), :] = z
        stats[0:1, :] += jnp.sum(z, axis=0, keepdims=True)
        stats[1:2, :] += jnp.sum(z * z, axis=0, keepdims=True)

    @pl.when(p == 1)
    def _():
        z = zbuf[pl.ds(t * ROWS, ROWS), :]
        m1 = stats[0:1, :] / nf
        v1 = stats[1:2, :] / nf - m1 * m1
        w = _leaky(_bn_apply(z, m1, v1, g1[...], be1[...])) * _RSN
        zbuf[pl.ds(t * ROWS, ROWS), :] = w
        stats[2:3, :] += jnp.sum(w, axis=0, keepdims=True)
        stats[3:4, :] += jnp.sum(w * w, axis=0, keepdims=True)

    @pl.when(p == 2)
    def _():
        w = zbuf[pl.ds(t * ROWS, ROWS), :]
        m2 = stats[2:3, :] / nf
        v2 = stats[3:4, :] / nf - m2 * m2
        out_ref[...] = _bn_apply(w, m2, v2, g2[...], be2[...])


def _tc_block0(x2, s2, epsb, w1, b1, w2, b2, g1, be1, g2, be2):
    vec_spec = pl.BlockSpec((1, H), lambda p, t: (0, 0))
    return pl.pallas_call(
        _tc_block0_body,
        grid=(3, TILES),
        in_specs=[
            pl.BlockSpec((ROWS, 1), lambda p, t: (t, 0)),
            pl.BlockSpec((2, ROWS, HALF), lambda p, t: (0, t, 0)),
            pl.BlockSpec((1, 1), lambda p, t: (0, 0)),
            vec_spec, vec_spec,
            pl.BlockSpec((H, H), lambda p, t: (0, 0)),
            vec_spec, vec_spec, vec_spec, vec_spec, vec_spec,
        ],
        out_specs=pl.BlockSpec((ROWS, H), lambda p, t: (t, 0)),
        out_shape=jax.ShapeDtypeStruct((N, H), jnp.float32),
        scratch_shapes=[
            pltpu.VMEM((N, H), jnp.float32),
            pltpu.VMEM((8, H), jnp.float32),
        ],
    )(x2, s2, epsb, w1, b1, w2, b2, g1, be1, g2, be2)


def _tc_final_body(h_ref, nb_ref, epsb, w1, b1, w2, b2, g1, be1, g2, be2,
                   l1w, l1b, l2w, l2b, out_ref, zbuf, ybuf, stats):
    p = pl.program_id(0)
    t = pl.program_id(1)
    nf = jnp.float32(N)

    @pl.when(jnp.logical_and(p == 0, t == 0))
    def _():
        stats[...] = jnp.zeros_like(stats)
        stats[4:5, :] = jnp.full((1, H), 3.0e38, jnp.float32)
        stats[5:6, :] = jnp.full((1, H), -3.0e38, jnp.float32)

    @pl.when(p == 0)
    def _():
        nb = jnp.concatenate([nb_ref[0], nb_ref[1]], axis=1)
        agg = h_ref[...] * epsb[...] + nb
        z = jnp.maximum(jnp.dot(agg, w1[...], preferred_element_type=jnp.float32,
                            precision=lax.Precision.HIGHEST)
                        + b1[...], 0.0)
        z = jnp.maximum(jnp.dot(z, w2[...], preferred_element_type=jnp.float32,
                            precision=lax.Precision.HIGHEST)
                        + b2[...], 0.0)
        zbuf[pl.ds(t * ROWS, ROWS), :] = z
        stats[0:1, :] += jnp.sum(z, axis=0, keepdims=True)
        stats[1:2, :] += jnp.sum(z * z, axis=0, keepdims=True)

    @pl.when(p == 1)
    def _():
        z = zbuf[pl.ds(t * ROWS, ROWS), :]
        m1 = stats[0:1, :] / nf
        v1 = stats[1:2, :] / nf - m1 * m1
        w = _leaky(_bn_apply(z, m1, v1, g1[...], be1[...])) * _RSN
        zbuf[pl.ds(t * ROWS, ROWS), :] = w
        stats[2:3, :] += jnp.sum(w, axis=0, keepdims=True)
        stats[3:4, :] += jnp.sum(w * w, axis=0, keepdims=True)

    @pl.when(p == 2)
    def _():
        w = zbuf[pl.ds(t * ROWS, ROWS), :]
        m2 = stats[2:3, :] / nf
        v2 = stats[3:4, :] / nf - m2 * m2
        hf = _bn_apply(w, m2, v2, g2[...], be2[...])
        y = _leaky(jnp.dot(hf, l1w[...], preferred_element_type=jnp.float32,
                            precision=lax.Precision.HIGHEST)
                   + l1b[...])
        y = _leaky(jnp.dot(y, l2w[...], preferred_element_type=jnp.float32,
                            precision=lax.Precision.HIGHEST)
                   + l2b[...])
        ybuf[pl.ds(t * ROWS, ROWS), :] = y
        stats[4:5, :] = jnp.minimum(stats[4:5, :], jnp.min(y))
        stats[5:6, :] = jnp.maximum(stats[5:6, :], jnp.max(y))

    @pl.when(p == 3)
    def _():
        y = ybuf[pl.ds(t * ROWS, ROWS), :]
        mn = stats[4:5, 0:1]
        mx = stats[5:6, 0:1]
        out_ref[...] = jnp.where(mn != mx, (y - mn) / (mx + 1e-6 - mn), y)


def _tc_final(h, nb2, epsb, w1, b1, w2, b2, g1, be1, g2, be2, l1w, l1b, l2w, l2b):
    vec_spec = pl.BlockSpec((1, H), lambda p, t: (0, 0))
    mat_spec = pl.BlockSpec((H, H), lambda p, t: (0, 0))
    return pl.pallas_call(
        _tc_final_body,
        grid=(4, TILES),
        in_specs=[
            pl.BlockSpec((ROWS, H), lambda p, t: (t, 0)),
            pl.BlockSpec((2, ROWS, HALF), lambda p, t: (0, t, 0)),
            vec_spec, mat_spec, vec_spec, mat_spec, vec_spec,
            vec_spec, vec_spec, vec_spec, vec_spec,
            pl.BlockSpec((H, 32), lambda p, t: (0, 0)),
            pl.BlockSpec((1, 32), lambda p, t: (0, 0)),
            pl.BlockSpec((32, 1), lambda p, t: (0, 0)),
            pl.BlockSpec((1, 1), lambda p, t: (0, 0)),
        ],
        out_specs=pl.BlockSpec((ROWS, 1), lambda p, t: (t, 0)),
        out_shape=jax.ShapeDtypeStruct((N, 1), jnp.float32),
        scratch_shapes=[
            pltpu.VMEM((N, H), jnp.float32),
            pltpu.VMEM((N, 1), jnp.float32),
            pltpu.VMEM((8, H), jnp.float32),
        ],
    )(h, nb2, epsb, w1, b1, w2, b2, g1, be1, g2, be2, l1w, l1b, l2w, l2b)


def kernel(x, edge_index, c0_eps, c0_w1, c0_b1, c0_w2, c0_b2, c0_g1, c0_be1,
           c0_g2, c0_be2, eps_s, w1_s, b1_s, w2_s, b2_s, g1_s, be1_s, g2_s,
           be2_s, lin1_w, lin1_b, lin2_w, lin2_b):
    x2 = x[:, None]
    x128 = jnp.broadcast_to(x2, (N, HALF))
    src = edge_index[0]
    dst = edge_index[1]

    s2 = _sc_segsum0(x128, src, dst)
    h = _tc_block0(
        x2, s2,
        (1.0 + c0_eps).reshape(1, 1),
        c0_w1.reshape(1, H), c0_b1.reshape(1, H),
        c0_w2, c0_b2.reshape(1, H),
        c0_g1.reshape(1, H), c0_be1.reshape(1, H),
        c0_g2.reshape(1, H), c0_be2.reshape(1, H),
    )

    for i in range(2):
        nb2 = _sc_segsum(h.reshape(2 * N, HALF), src, dst)
        h = _tc_block(
            h, nb2,
            jnp.broadcast_to((1.0 + eps_s[i]).reshape(1, 1), (1, H)),
            w1_s[i], b1_s[i].reshape(1, H), w2_s[i], b2_s[i].reshape(1, H),
            g1_s[i].reshape(1, H), be1_s[i].reshape(1, H),
            g2_s[i].reshape(1, H), be2_s[i].reshape(1, H),
        )

    nb2 = _sc_segsum(h.reshape(2 * N, HALF), src, dst)
    return _tc_final(
        h, nb2,
        jnp.broadcast_to((1.0 + eps_s[2]).reshape(1, 1), (1, H)),
        w1_s[2], b1_s[2].reshape(1, H), w2_s[2], b2_s[2].reshape(1, H),
        g1_s[2].reshape(1, H), be1_s[2].reshape(1, H),
        g2_s[2].reshape(1, H), be2_s[2].reshape(1, H),
        lin1_w, lin1_b.reshape(1, 32), lin2_w, lin2_b.reshape(1, 1),
    )


# speedup vs baseline: 1.0136x; 1.0136x over previous
import functools

import jax
import jax.numpy as jnp
from jax import lax
from jax.experimental import pallas as pl
from jax.experimental.pallas import tpu as pltpu
from jax.experimental.pallas import tpu_sc as plsc

N = 10000
E = 160000
H = 256
HALF = 128
ROWS = 1000
TILES = N // ROWS
NSUB = 16
RPS = 640
NPAD = NSUB * RPS

@functools.cache
def _sc_mesh():
    return plsc.VectorSubcoreMesh(core_axis_name="c", subcore_axis_name="s")


_CHUNK = 96


def _make_sc_body(nchunks, tail, feature_split, eps_sub):
    def body(tbl_hbm, src_hbm, dst_hbm, out_hbm, srcv2, dstv2, gv2, rows2,
             srcT, dstT, gvT, rowsT, zv, acc, sem_i, sem_g0, sem_g1):
        c = lax.axis_index("c")
        s = lax.axis_index("s")

        @pl.loop(0, 64)
        def _(i):
            @pl.loop(0, HALF, step=16)
            def _(j):
                zv[i, pl.ds(j, 16)] = jnp.zeros((16,), jnp.float32)

        base_r = s * RPS

        @pl.loop(0, RPS // 64)
        def _(k):
            pltpu.sync_copy(zv, acc.at[pl.ds(base_r + k * 64, 64)])

        plsc.subcore_barrier()

        if feature_split:
            base_e = s * eps_sub
        else:
            base_e = (c * NSUB + s) * eps_sub

        gsems = (sem_g0, sem_g1)

        def fetch_idx(k, slot):
            off = base_e + k * _CHUNK
            a = pltpu.async_copy(src_hbm.at[pl.ds(off, _CHUNK)],
                                 srcv2.at[slot], sem_i)
            b = pltpu.async_copy(dst_hbm.at[pl.ds(off, _CHUNK)],
                                 dstv2.at[slot], sem_i)
            a.wait()
            b.wait()
            if feature_split:
                @pl.loop(0, _CHUNK, step=16)
                def _(j):
                    gv2[slot, pl.ds(j, 16)] = srcv2[slot, pl.ds(j, 16)] * 2 + c

        def gidx(slot):
            return gv2.at[slot] if feature_split else srcv2.at[slot]

        def fire_gather(slot):
            pltpu.async_copy(tbl_hbm.at[gidx(slot)], rows2.at[slot],
                             gsems[slot])

        def wait_gather(slot):
            pltpu.make_async_copy(tbl_hbm.at[gidx(slot)], rows2.at[slot],
                                  gsems[slot]).wait()

        def scatter(slot):
            pltpu.sync_copy(rows2.at[slot], acc.at[dstv2.at[slot]], add=True)

        def step(kcur, cur, nxt):
            @pl.when(kcur + 1 < nchunks)
            def _():
                fetch_idx(kcur + 1, nxt)

            @pl.when(kcur < nchunks)
            def _():
                wait_gather(cur)

            @pl.when(kcur + 1 < nchunks)
            def _():
                fire_gather(nxt)

            @pl.when(kcur < nchunks)
            def _():
                scatter(cur)

        fetch_idx(0, 0)
        fire_gather(0)

        @pl.loop(0, (nchunks + 1) // 2)
        def _(i):
            step(2 * i, 0, 1)
            step(2 * i + 1, 1, 0)

        if tail:
            off = base_e + nchunks * _CHUNK
            a = pltpu.async_copy(src_hbm.at[pl.ds(off, tail)], srcT, sem_i)
            b = pltpu.async_copy(dst_hbm.at[pl.ds(off, tail)], dstT, sem_i)
            a.wait()
            b.wait()
            if feature_split:
                @pl.loop(0, tail, step=16)
                def _(j):
                    gvT[pl.ds(j, 16)] = srcT[pl.ds(j, 16)] * 2 + c
                pltpu.async_copy(tbl_hbm.at[gvT], rowsT, sem_g0).wait()
            else:
                pltpu.async_copy(tbl_hbm.at[srcT], rowsT, sem_g0).wait()
            pltpu.sync_copy(rowsT, acc.at[dstT], add=True)

        plsc.subcore_barrier()
        pltpu.sync_copy(acc.at[pl.ds(base_r, RPS)],
                        out_hbm.at[c, pl.ds(base_r, RPS)])

    return body


def _sc_call(body, tail, tbl, src, dst):
    kern = pl.kernel(
        body,
        out_type=jax.ShapeDtypeStruct((2, NPAD, HALF), jnp.float32),
        mesh=_sc_mesh(),
        scratch_types=[
            pltpu.VMEM((2, _CHUNK), jnp.int32),
            pltpu.VMEM((2, _CHUNK), jnp.int32),
            pltpu.VMEM((2, _CHUNK), jnp.int32),
            pltpu.VMEM((2, _CHUNK, HALF), jnp.float32),
            pltpu.VMEM((tail,), jnp.int32),
            pltpu.VMEM((tail,), jnp.int32),
            pltpu.VMEM((tail,), jnp.int32),
            pltpu.VMEM((tail, HALF), jnp.float32),
            pltpu.VMEM((64, HALF), jnp.float32),
            pltpu.VMEM_SHARED((NPAD, HALF), jnp.float32),
            pltpu.SemaphoreType.DMA,
            pltpu.SemaphoreType.DMA,
            pltpu.SemaphoreType.DMA,
        ],
    )
    return kern(tbl, src, dst)


_EPS_SUB = E // NSUB
_EPS_SUB0 = E // (2 * NSUB)
_MAIN_BODY = _make_sc_body(_EPS_SUB // _CHUNK, _EPS_SUB % _CHUNK, True,
                           _EPS_SUB)
_BLK0_BODY = _make_sc_body(_EPS_SUB0 // _CHUNK, _EPS_SUB0 % _CHUNK, False,
                           _EPS_SUB0)


def _sc_segsum(h2, src, dst):
    return _sc_call(_MAIN_BODY, _EPS_SUB % _CHUNK, h2, src, dst)


def _sc_segsum0(x128, src, dst):
    return _sc_call(_BLK0_BODY, _EPS_SUB0 % _CHUNK, x128, src, dst)


_RSN = 1.0 / float(N) ** 0.5


def _bn_apply(v, mrow, vrow, g, b):
    return (v - mrow) * lax.rsqrt(vrow + 1e-5) * g + b


def _leaky(v):
    return jnp.where(v >= 0, v, 0.01 * v)


def _tc_block_body(h_ref, nb_ref, epsb, w1, b1, w2, b2, g1, be1, g2, be2,
                   out_ref, zbuf, stats):
    p = pl.program_id(0)
    t = pl.program_id(1)
    nf = jnp.float32(N)

    @pl.when(jnp.logical_and(p == 0, t == 0))
    def _():
        stats[...] = jnp.zeros_like(stats)

    @pl.when(p == 0)
    def _():
        nb = jnp.concatenate([nb_ref[0], nb_ref[1]], axis=1)
        agg = h_ref[...] * epsb[...] + nb
        z = jnp.maximum(jnp.dot(agg, w1[...], preferred_element_type=jnp.float32,
                            precision=lax.Precision.HIGHEST)
                        + b1[...], 0.0)
        z = jnp.maximum(jnp.dot(z, w2[...], preferred_element_type=jnp.float32,
                            precision=lax.Precision.HIGHEST)
                        + b2[...], 0.0)
        zbuf[pl.ds(t * ROWS, ROWS), :] = z
        stats[0:1, :] += jnp.sum(z, axis=0, keepdims=True)
        stats[1:2, :] += jnp.sum(z * z, axis=0, keepdims=True)

    @pl.when(p == 1)
    def _():
        z = zbuf[pl.ds(t * ROWS, ROWS), :]
        m1 = stats[0:1, :] / nf
        v1 = stats[1:2, :] / nf - m1 * m1
        w = _leaky(_bn_apply(z, m1, v1, g1[...], be1[...])) * _RSN
        zbuf[pl.ds(t * ROWS, ROWS), :] = w
        stats[2:3, :] += jnp.sum(w, axis=0, keepdims=True)
        stats[3:4, :] += jnp.sum(w * w, axis=0, keepdims=True)

    @pl.when(p == 2)
    def _():
        w = zbuf[pl.ds(t * ROWS, ROWS), :]
        m2 = stats[2:3, :] / nf
        v2 = stats[3:4, :] / nf - m2 * m2
        out_ref[...] = _bn_apply(w, m2, v2, g2[...], be2[...])


def _tc_block(h, nb2, epsb, w1, b1, w2, b2, g1, be1, g2, be2):
    row_spec = pl.BlockSpec((ROWS, H), lambda p, t: (t, 0))
    vec_spec = pl.BlockSpec((1, H), lambda p, t: (0, 0))
    mat_spec = pl.BlockSpec((H, H), lambda p, t: (0, 0))
    return pl.pallas_call(
        _tc_block_body,
        grid=(3, TILES),
        in_specs=[
            row_spec,
            pl.BlockSpec((2, ROWS, HALF), lambda p, t: (0, t, 0)),
            vec_spec, mat_spec, vec_spec, mat_spec, vec_spec,
            vec_spec, vec_spec, vec_spec, vec_spec,
        ],
        out_specs=row_spec,
        out_shape=jax.ShapeDtypeStruct((N, H), jnp.float32),
        scratch_shapes=[
            pltpu.VMEM((N, H), jnp.float32),
            pltpu.VMEM((8, H), jnp.float32),
        ],
    )(h, nb2, epsb, w1, b1, w2, b2, g1, be1, g2, be2)


def _tc_block0_body(x_ref, s2_ref, epsb, w1, b1, w2, b2, g1, be1, g2, be2,
                    out_ref, zbuf, stats):
    p = pl.program_id(0)
    t = pl.program_id(1)
    nf = jnp.float32(N)

    @pl.when(jnp.logical_and(p == 0, t == 0))
    def _():
        stats[...] = jnp.zeros_like(stats)

    @pl.when(p == 0)
    def _():
        s = s2_ref[0] + s2_ref[1]
        agg = x_ref[...] * epsb[...] + s
        z = jnp.maximum(agg * w1[...] + b1[...], 0.0)
        z = jnp.maximum(jnp.dot(z, w2[...], preferred_element_type=jnp.float32,
                            precision=lax.Precision.HIGHEST)
                        + b2[...], 0.0)
        zbuf[pl.ds(t * ROWS, ROWS), :] = z
        stats[0:1, :] += jnp.sum(z, axis=0, keepdims=True)
        stats[1:2, :] += jnp.sum(z * z, axis=0, keepdims=True)

    @pl.when(p == 1)
    def _():
        z = zbuf[pl.ds(t * ROWS, ROWS), :]
        m1 = stats[0:1, :] / nf
        v1 = stats[1:2, :] / nf - m1 * m1
        w = _leaky(_bn_apply(z, m1, v1, g1[...], be1[...])) * _RSN
        zbuf[pl.ds(t * ROWS, ROWS), :] = w
        stats[2:3, :] += jnp.sum(w, axis=0, keepdims=True)
        stats[3:4, :] += jnp.sum(w * w, axis=0, keepdims=True)

    @pl.when(p == 2)
    def _():
        w = zbuf[pl.ds(t * ROWS, ROWS), :]
        m2 = stats[2:3, :] / nf
        v2 = stats[3:4, :] / nf - m2 * m2
        out_ref[...] = _bn_apply(w, m2, v2, g2[...], be2[...])


def _tc_block0(x2, s2, epsb, w1, b1, w2, b2, g1, be1, g2, be2):
    vec_spec = pl.BlockSpec((1, H), lambda p, t: (0, 0))
    return pl.pallas_call(
        _tc_block0_body,
        grid=(3, TILES),
        in_specs=[
            pl.BlockSpec((ROWS, 1), lambda p, t: (t, 0)),
            pl.BlockSpec((2, ROWS, 1), lambda p, t: (0, t, 0)),
            pl.BlockSpec((1, 1), lambda p, t: (0, 0)),
            vec_spec, vec_spec,
            pl.BlockSpec((H, H), lambda p, t: (0, 0)),
            vec_spec, vec_spec, vec_spec, vec_spec, vec_spec,
        ],
        out_specs=pl.BlockSpec((ROWS, H), lambda p, t: (t, 0)),
        out_shape=jax.ShapeDtypeStruct((N, H), jnp.float32),
        scratch_shapes=[
            pltpu.VMEM((N, H), jnp.float32),
            pltpu.VMEM((8, H), jnp.float32),
        ],
    )(x2, s2, epsb, w1, b1, w2, b2, g1, be1, g2, be2)


def _tc_final_body(h_ref, nb_ref, epsb, w1, b1, w2, b2, g1, be1, g2, be2,
                   l1w, l1b, l2w, l2b, out_ref, zbuf, ybuf, stats):
    p = pl.program_id(0)
    t = pl.program_id(1)
    nf = jnp.float32(N)

    @pl.when(jnp.logical_and(p == 0, t == 0))
    def _():
        stats[...] = jnp.zeros_like(stats)
        stats[4:5, :] = jnp.full((1, H), 3.0e38, jnp.float32)
        stats[5:6, :] = jnp.full((1, H), -3.0e38, jnp.float32)

    @pl.when(p == 0)
    def _():
        nb = jnp.concatenate([nb_ref[0], nb_ref[1]], axis=1)
        agg = h_ref[...] * epsb[...] + nb
        z = jnp.maximum(jnp.dot(agg, w1[...], preferred_element_type=jnp.float32,
                            precision=lax.Precision.HIGHEST)
                        + b1[...], 0.0)
        z = jnp.maximum(jnp.dot(z, w2[...], preferred_element_type=jnp.float32,
                            precision=lax.Precision.HIGHEST)
                        + b2[...], 0.0)
        zbuf[pl.ds(t * ROWS, ROWS), :] = z
        stats[0:1, :] += jnp.sum(z, axis=0, keepdims=True)
        stats[1:2, :] += jnp.sum(z * z, axis=0, keepdims=True)

    @pl.when(p == 1)
    def _():
        z = zbuf[pl.ds(t * ROWS, ROWS), :]
        m1 = stats[0:1, :] / nf
        v1 = stats[1:2, :] / nf - m1 * m1
        w = _leaky(_bn_apply(z, m1, v1, g1[...], be1[...])) * _RSN
        zbuf[pl.ds(t * ROWS, ROWS), :] = w
        stats[2:3, :] += jnp.sum(w, axis=0, keepdims=True)
        stats[3:4, :] += jnp.sum(w * w, axis=0, keepdims=True)

    @pl.when(p == 2)
    def _():
        w = zbuf[pl.ds(t * ROWS, ROWS), :]
        m2 = stats[2:3, :] / nf
        v2 = stats[3:4, :] / nf - m2 * m2
        hf = _bn_apply(w, m2, v2, g2[...], be2[...])
        y = _leaky(jnp.dot(hf, l1w[...], preferred_element_type=jnp.float32,
                            precision=lax.Precision.HIGHEST)
                   + l1b[...])
        y = _leaky(jnp.dot(y, l2w[...], preferred_element_type=jnp.float32,
                            precision=lax.Precision.HIGHEST)
                   + l2b[...])
        ybuf[pl.ds(t * ROWS, ROWS), :] = y
        stats[4:5, :] = jnp.minimum(stats[4:5, :], jnp.min(y))
        stats[5:6, :] = jnp.maximum(stats[5:6, :], jnp.max(y))

    @pl.when(p == 3)
    def _():
        y = ybuf[pl.ds(t * ROWS, ROWS), :]
        mn = stats[4:5, 0:1]
        mx = stats[5:6, 0:1]
        out_ref[...] = jnp.where(mn != mx, (y - mn) / (mx + 1e-6 - mn), y)


def _tc_final(h, nb2, epsb, w1, b1, w2, b2, g1, be1, g2, be2, l1w, l1b, l2w, l2b):
    vec_spec = pl.BlockSpec((1, H), lambda p, t: (0, 0))
    mat_spec = pl.BlockSpec((H, H), lambda p, t: (0, 0))
    return pl.pallas_call(
        _tc_final_body,
        grid=(4, TILES),
        in_specs=[
            pl.BlockSpec((ROWS, H), lambda p, t: (t, 0)),
            pl.BlockSpec((2, ROWS, HALF), lambda p, t: (0, t, 0)),
            vec_spec, mat_spec, vec_spec, mat_spec, vec_spec,
            vec_spec, vec_spec, vec_spec, vec_spec,
            pl.BlockSpec((H, 32), lambda p, t: (0, 0)),
            pl.BlockSpec((1, 32), lambda p, t: (0, 0)),
            pl.BlockSpec((32, 1), lambda p, t: (0, 0)),
            pl.BlockSpec((1, 1), lambda p, t: (0, 0)),
        ],
        out_specs=pl.BlockSpec((ROWS, 1), lambda p, t: (t, 0)),
        out_shape=jax.ShapeDtypeStruct((N, 1), jnp.float32),
        scratch_shapes=[
            pltpu.VMEM((N, H), jnp.float32),
            pltpu.VMEM((N, 1), jnp.float32),
            pltpu.VMEM((8, H), jnp.float32),
        ],
    )(h, nb2, epsb, w1, b1, w2, b2, g1, be1, g2, be2, l1w, l1b, l2w, l2b)


def kernel(x, edge_index, c0_eps, c0_w1, c0_b1, c0_w2, c0_b2, c0_g1, c0_be1,
           c0_g2, c0_be2, eps_s, w1_s, b1_s, w2_s, b2_s, g1_s, be1_s, g2_s,
           be2_s, lin1_w, lin1_b, lin2_w, lin2_b):
    x2 = x[:, None]
    x128 = jnp.broadcast_to(x2, (N, HALF))
    src = edge_index[0]
    dst = edge_index[1]

    s2 = _sc_segsum0(x128, src, dst)[:, :N, :1]
    h = _tc_block0(
        x2, s2,
        (1.0 + c0_eps).reshape(1, 1),
        c0_w1.reshape(1, H), c0_b1.reshape(1, H),
        c0_w2, c0_b2.reshape(1, H),
        c0_g1.reshape(1, H), c0_be1.reshape(1, H),
        c0_g2.reshape(1, H), c0_be2.reshape(1, H),
    )

    for i in range(2):
        nb2 = _sc_segsum(h.reshape(2 * N, HALF), src, dst)[:, :N]
        h = _tc_block(
            h, nb2,
            jnp.broadcast_to((1.0 + eps_s[i]).reshape(1, 1), (1, H)),
            w1_s[i], b1_s[i].reshape(1, H), w2_s[i], b2_s[i].reshape(1, H),
            g1_s[i].reshape(1, H), be1_s[i].reshape(1, H),
            g2_s[i].reshape(1, H), be2_s[i].reshape(1, H),
        )

    nb2 = _sc_segsum(h.reshape(2 * N, HALF), src, dst)[:, :N]
    return _tc_final(
        h, nb2,
        jnp.broadcast_to((1.0 + eps_s[2]).reshape(1, 1), (1, H)),
        w1_s[2], b1_s[2].reshape(1, H), w2_s[2], b2_s[2].reshape(1, H),
        g1_s[2].reshape(1, H), be1_s[2].reshape(1, H),
        g2_s[2].reshape(1, H), be2_s[2].reshape(1, H),
        lin1_w, lin1_b.reshape(1, 32), lin2_w, lin2_b.reshape(1, 1),
    )


# speedup vs baseline: 1.2017x; 1.1855x over previous
import functools

import jax
import jax.numpy as jnp
from jax import lax
from jax.experimental import pallas as pl
from jax.experimental.pallas import tpu as pltpu
from jax.experimental.pallas import tpu_sc as plsc

N = 10000
E = 160000
H = 256
HALF = 128
ROWS = 1000
TILES = N // ROWS
NSUB = 16
RPS = 640
NPAD = NSUB * RPS

@functools.cache
def _sc_mesh():
    return plsc.VectorSubcoreMesh(core_axis_name="c", subcore_axis_name="s")


_CHUNK = 96


def _make_sc_body(nchunks, tail, feature_split, eps_sub):
    def body(tbl_hbm, src_hbm, dst_hbm, out_hbm, srcv2, dstv2, gv2, rows2,
             srcT, dstT, gvT, rowsT, zv, acc, sem_i, sem_g0, sem_g1):
        c = lax.axis_index("c")
        s = lax.axis_index("s")

        @pl.loop(0, 64)
        def _(i):
            @pl.loop(0, HALF, step=16)
            def _(j):
                zv[i, pl.ds(j, 16)] = jnp.zeros((16,), jnp.float32)

        base_r = s * RPS

        @pl.loop(0, RPS // 64)
        def _(k):
            pltpu.sync_copy(zv, acc.at[pl.ds(base_r + k * 64, 64)])

        plsc.subcore_barrier()

        if feature_split:
            base_e = s * eps_sub
        else:
            base_e = (c * NSUB + s) * eps_sub

        gsems = (sem_g0, sem_g1)

        def fetch_idx(k, slot):
            off = base_e + k * _CHUNK
            a = pltpu.async_copy(src_hbm.at[pl.ds(off, _CHUNK)],
                                 srcv2.at[slot], sem_i)
            b = pltpu.async_copy(dst_hbm.at[pl.ds(off, _CHUNK)],
                                 dstv2.at[slot], sem_i)
            a.wait()
            b.wait()
            if feature_split:
                @pl.loop(0, _CHUNK, step=16)
                def _(j):
                    gv2[slot, pl.ds(j, 16)] = srcv2[slot, pl.ds(j, 16)] * 2 + c

        def gidx(slot):
            return gv2.at[slot] if feature_split else srcv2.at[slot]

        def fire_gather(slot):
            pltpu.async_copy(tbl_hbm.at[gidx(slot)], rows2.at[slot],
                             gsems[slot])

        def wait_gather(slot):
            pltpu.make_async_copy(tbl_hbm.at[gidx(slot)], rows2.at[slot],
                                  gsems[slot]).wait()

        def scatter(slot):
            pltpu.sync_copy(rows2.at[slot], acc.at[dstv2.at[slot]], add=True)

        def step(kcur, cur, nxt):
            @pl.when(kcur + 1 < nchunks)
            def _():
                fetch_idx(kcur + 1, nxt)

            @pl.when(kcur < nchunks)
            def _():
                wait_gather(cur)

            @pl.when(kcur + 1 < nchunks)
            def _():
                fire_gather(nxt)

            @pl.when(kcur < nchunks)
            def _():
                scatter(cur)

        fetch_idx(0, 0)
        fire_gather(0)

        @pl.loop(0, (nchunks + 1) // 2)
        def _(i):
            step(2 * i, 0, 1)
            step(2 * i + 1, 1, 0)

        if tail:
            off = base_e + nchunks * _CHUNK
            a = pltpu.async_copy(src_hbm.at[pl.ds(off, tail)], srcT, sem_i)
            b = pltpu.async_copy(dst_hbm.at[pl.ds(off, tail)], dstT, sem_i)
            a.wait()
            b.wait()
            if feature_split:
                @pl.loop(0, tail, step=16)
                def _(j):
                    gvT[pl.ds(j, 16)] = srcT[pl.ds(j, 16)] * 2 + c
                pltpu.async_copy(tbl_hbm.at[gvT], rowsT, sem_g0).wait()
            else:
                pltpu.async_copy(tbl_hbm.at[srcT], rowsT, sem_g0).wait()
            pltpu.sync_copy(rowsT, acc.at[dstT], add=True)

        plsc.subcore_barrier()
        pltpu.sync_copy(acc.at[pl.ds(base_r, RPS)],
                        out_hbm.at[c, pl.ds(base_r, RPS)])

    return body


def _sc_call(body, tail, tbl, src, dst):
    kern = pl.kernel(
        body,
        out_type=jax.ShapeDtypeStruct((2, NPAD, HALF), jnp.float32),
        mesh=_sc_mesh(),
        scratch_types=[
            pltpu.VMEM((2, _CHUNK), jnp.int32),
            pltpu.VMEM((2, _CHUNK), jnp.int32),
            pltpu.VMEM((2, _CHUNK), jnp.int32),
            pltpu.VMEM((2, _CHUNK, HALF), jnp.float32),
            pltpu.VMEM((tail,), jnp.int32),
            pltpu.VMEM((tail,), jnp.int32),
            pltpu.VMEM((tail,), jnp.int32),
            pltpu.VMEM((tail, HALF), jnp.float32),
            pltpu.VMEM((64, HALF), jnp.float32),
            pltpu.VMEM_SHARED((NPAD, HALF), jnp.float32),
            pltpu.SemaphoreType.DMA,
            pltpu.SemaphoreType.DMA,
            pltpu.SemaphoreType.DMA,
        ],
    )
    return kern(tbl, src, dst)


_EPS_SUB = E // NSUB
_EPS_SUB0 = E // (2 * NSUB)
_MAIN_BODY = _make_sc_body(_EPS_SUB // _CHUNK, _EPS_SUB % _CHUNK, True,
                           _EPS_SUB)
_BLK0_BODY = _make_sc_body(_EPS_SUB0 // _CHUNK, _EPS_SUB0 % _CHUNK, False,
                           _EPS_SUB0)


def _sc_segsum(h2, src, dst):
    return _sc_call(_MAIN_BODY, _EPS_SUB % _CHUNK, h2, src, dst)


def _sc_segsum0(x128, src, dst):
    return _sc_call(_BLK0_BODY, _EPS_SUB0 % _CHUNK, x128, src, dst)


_RSN = 1.0 / float(N) ** 0.5


def _bn_apply(v, mrow, vrow, g, b):
    return (v - mrow) * lax.rsqrt(vrow + 1e-5) * g + b


def _leaky(v):
    return jnp.where(v >= 0, v, 0.01 * v)


def _gating_spec(shape):
    nd = len(shape)
    if nd == 2:
        return pl.BlockSpec(shape, lambda p, t: (jnp.where(p == 0, t, 0), 0))
    return pl.BlockSpec(shape, lambda p, t: (0, jnp.where(p == 0, t, 0), 0))


def _tc_block_body(h_ref, nb_ref, epsb, w1, b1, w2, b2, g1, be1, g2, be2,
                   out_ref, zbuf, stats):
    p = pl.program_id(0)
    t = pl.program_id(1)
    nf = jnp.float32(N)
    rows = pl.ds(t * ROWS, ROWS)

    @pl.when(jnp.logical_and(p == 0, t == 0))
    def _():
        stats[...] = jnp.zeros_like(stats)

    @pl.when(p == 0)
    def _():
        nb = jnp.concatenate([nb_ref[0], nb_ref[1]], axis=1)
        agg = h_ref[...] * epsb[...] + nb
        z = jnp.maximum(jnp.dot(agg, w1[...], preferred_element_type=jnp.float32)
                        + b1[...], 0.0)
        z = jnp.maximum(jnp.dot(z, w2[...], preferred_element_type=jnp.float32)
                        + b2[...], 0.0)
        zbuf[rows, :] = z
        stats[0:1, :] += jnp.sum(z, axis=0, keepdims=True)

    @pl.when(p == 1)
    def _():
        z = zbuf[rows, :]
        m1 = stats[0:1, :] / nf
        d = z - m1
        stats[1:2, :] += jnp.sum(d * d, axis=0, keepdims=True)

    @pl.when(p == 2)
    def _():
        z = zbuf[rows, :]
        m1 = stats[0:1, :] / nf
        sd1 = jnp.sqrt(stats[1:2, :] / nf + 1e-5)
        zn = (z - m1) / sd1 * g1[...] + be1[...]
        w = jnp.where(zn >= 0, zn, 0.01 * zn) * _RSN
        zbuf[rows, :] = w
        stats[2:3, :] += jnp.sum(w, axis=0, keepdims=True)

    @pl.when(p == 3)
    def _():
        w = zbuf[rows, :]
        m2 = stats[2:3, :] / nf
        d = w - m2
        stats[3:4, :] += jnp.sum(d * d, axis=0, keepdims=True)

    @pl.when(p == 4)
    def _():
        w = zbuf[rows, :]
        m2 = stats[2:3, :] / nf
        sd2 = jnp.sqrt(stats[3:4, :] / nf + 1e-5)
        out_ref[...] = (w - m2) / sd2 * g2[...] + be2[...]


def _tc_block(h, nb2, epsb, w1, b1, w2, b2, g1, be1, g2, be2):
    vec_spec = pl.BlockSpec((1, H), lambda p, t: (0, 0))
    mat_spec = pl.BlockSpec((H, H), lambda p, t: (0, 0))
    return pl.pallas_call(
        _tc_block_body,
        grid=(5, TILES),
        in_specs=[
            _gating_spec((ROWS, H)),
            _gating_spec((2, ROWS, HALF)),
            vec_spec, mat_spec, vec_spec, mat_spec, vec_spec,
            vec_spec, vec_spec, vec_spec, vec_spec,
        ],
        out_specs=pl.BlockSpec((ROWS, H), lambda p, t: (jnp.where(p == 4, t, 0), 0)),
        out_shape=jax.ShapeDtypeStruct((N, H), jnp.float32),
        scratch_shapes=[
            pltpu.VMEM((N, H), jnp.float32),
            pltpu.VMEM((8, H), jnp.float32),
        ],
    )(h, nb2, epsb, w1, b1, w2, b2, g1, be1, g2, be2)


def _tc_block0_body(x_ref, s2_ref, epsb, w1, b1, w2, b2, g1, be1, g2, be2,
                    out_ref, zbuf, stats):
    p = pl.program_id(0)
    t = pl.program_id(1)
    nf = jnp.float32(N)
    rows = pl.ds(t * ROWS, ROWS)

    @pl.when(jnp.logical_and(p == 0, t == 0))
    def _():
        stats[...] = jnp.zeros_like(stats)

    @pl.when(p == 0)
    def _():
        s = s2_ref[0] + s2_ref[1]
        agg = x_ref[...] * epsb[...] + s
        z = jnp.maximum(agg * w1[...] + b1[...], 0.0)
        z = jnp.maximum(jnp.dot(z, w2[...], preferred_element_type=jnp.float32)
                        + b2[...], 0.0)
        zbuf[rows, :] = z
        stats[0:1, :] += jnp.sum(z, axis=0, keepdims=True)

    @pl.when(p == 1)
    def _():
        z = zbuf[rows, :]
        m1 = stats[0:1, :] / nf
        d = z - m1
        stats[1:2, :] += jnp.sum(d * d, axis=0, keepdims=True)

    @pl.when(p == 2)
    def _():
        z = zbuf[rows, :]
        m1 = stats[0:1, :] / nf
        sd1 = jnp.sqrt(stats[1:2, :] / nf + 1e-5)
        zn = (z - m1) / sd1 * g1[...] + be1[...]
        w = jnp.where(zn >= 0, zn, 0.01 * zn) * _RSN
        zbuf[rows, :] = w
        stats[2:3, :] += jnp.sum(w, axis=0, keepdims=True)

    @pl.when(p == 3)
    def _():
        w = zbuf[rows, :]
        m2 = stats[2:3, :] / nf
        d = w - m2
        stats[3:4, :] += jnp.sum(d * d, axis=0, keepdims=True)

    @pl.when(p == 4)
    def _():
        w = zbuf[rows, :]
        m2 = stats[2:3, :] / nf
        sd2 = jnp.sqrt(stats[3:4, :] / nf + 1e-5)
        out_ref[...] = (w - m2) / sd2 * g2[...] + be2[...]


def _tc_block0(x2, s2, epsb, w1, b1, w2, b2, g1, be1, g2, be2):
    vec_spec = pl.BlockSpec((1, H), lambda p, t: (0, 0))
    return pl.pallas_call(
        _tc_block0_body,
        grid=(5, TILES),
        in_specs=[
            _gating_spec((ROWS, 1)),
            _gating_spec((2, ROWS, 1)),
            pl.BlockSpec((1, 1), lambda p, t: (0, 0)),
            vec_spec, vec_spec,
            pl.BlockSpec((H, H), lambda p, t: (0, 0)),
            vec_spec, vec_spec, vec_spec, vec_spec, vec_spec,
        ],
        out_specs=pl.BlockSpec((ROWS, H), lambda p, t: (jnp.where(p == 4, t, 0), 0)),
        out_shape=jax.ShapeDtypeStruct((N, H), jnp.float32),
        scratch_shapes=[
            pltpu.VMEM((N, H), jnp.float32),
            pltpu.VMEM((8, H), jnp.float32),
        ],
    )(x2, s2, epsb, w1, b1, w2, b2, g1, be1, g2, be2)


def _tc_final_body(h_ref, nb_ref, epsb, w1, b1, w2, b2, g1, be1, g2, be2,
                   l1w, l1b, l2w, l2b, out_ref, zbuf, ybuf, stats):
    p = pl.program_id(0)
    t = pl.program_id(1)
    nf = jnp.float32(N)
    rows = pl.ds(t * ROWS, ROWS)

    @pl.when(jnp.logical_and(p == 0, t == 0))
    def _():
        stats[...] = jnp.zeros_like(stats)
        stats[4:5, :] = jnp.full((1, H), 3.0e38, jnp.float32)
        stats[5:6, :] = jnp.full((1, H), -3.0e38, jnp.float32)

    @pl.when(p == 0)
    def _():
        nb = jnp.concatenate([nb_ref[0], nb_ref[1]], axis=1)
        agg = h_ref[...] * epsb[...] + nb
        z = jnp.maximum(jnp.dot(agg, w1[...], preferred_element_type=jnp.float32)
                        + b1[...], 0.0)
        z = jnp.maximum(jnp.dot(z, w2[...], preferred_element_type=jnp.float32)
                        + b2[...], 0.0)
        zbuf[rows, :] = z
        stats[0:1, :] += jnp.sum(z, axis=0, keepdims=True)

    @pl.when(p == 1)
    def _():
        z = zbuf[rows, :]
        m1 = stats[0:1, :] / nf
        d = z - m1
        stats[1:2, :] += jnp.sum(d * d, axis=0, keepdims=True)

    @pl.when(p == 2)
    def _():
        z = zbuf[rows, :]
        m1 = stats[0:1, :] / nf
        sd1 = jnp.sqrt(stats[1:2, :] / nf + 1e-5)
        zn = (z - m1) / sd1 * g1[...] + be1[...]
        w = jnp.where(zn >= 0, zn, 0.01 * zn) * _RSN
        zbuf[rows, :] = w
        stats[2:3, :] += jnp.sum(w, axis=0, keepdims=True)

    @pl.when(p == 3)
    def _():
        w = zbuf[rows, :]
        m2 = stats[2:3, :] / nf
        d = w - m2
        stats[3:4, :] += jnp.sum(d * d, axis=0, keepdims=True)

    @pl.when(p == 4)
    def _():
        w = zbuf[rows, :]
        m2 = stats[2:3, :] / nf
        sd2 = jnp.sqrt(stats[3:4, :] / nf + 1e-5)
        hf = (w - m2) / sd2 * g2[...] + be2[...]
        y = jnp.dot(hf, l1w[...], preferred_element_type=jnp.float32) + l1b[...]
        y = jnp.where(y >= 0, y, 0.01 * y)
        y = jnp.dot(y, l2w[...], preferred_element_type=jnp.float32) + l2b[...]
        y = jnp.where(y >= 0, y, 0.01 * y)
        ybuf[rows, :] = y
        stats[4:5, :] = jnp.minimum(stats[4:5, :], jnp.min(y))
        stats[5:6, :] = jnp.maximum(stats[5:6, :], jnp.max(y))

    @pl.when(p == 5)
    def _():
        y = ybuf[rows, :]
        mn = stats[4:5, 0:1]
        mx = stats[5:6, 0:1]
        out_ref[...] = jnp.where(mn != mx, (y - mn) / (mx + 1e-6 - mn), y)


def _tc_final(h, nb2, epsb, w1, b1, w2, b2, g1, be1, g2, be2, l1w, l1b, l2w, l2b):
    vec_spec = pl.BlockSpec((1, H), lambda p, t: (0, 0))
    mat_spec = pl.BlockSpec((H, H), lambda p, t: (0, 0))
    return pl.pallas_call(
        _tc_final_body,
        grid=(6, TILES),
        in_specs=[
            _gating_spec((ROWS, H)),
            _gating_spec((2, ROWS, HALF)),
            vec_spec, mat_spec, vec_spec, mat_spec, vec_spec,
            vec_spec, vec_spec, vec_spec, vec_spec,
            pl.BlockSpec((H, 32), lambda p, t: (0, 0)),
            pl.BlockSpec((1, 32), lambda p, t: (0, 0)),
            pl.BlockSpec((32, 1), lambda p, t: (0, 0)),
            pl.BlockSpec((1, 1), lambda p, t: (0, 0)),
        ],
        out_specs=pl.BlockSpec((ROWS, 1), lambda p, t: (jnp.where(p == 5, t, 0), 0)),
        out_shape=jax.ShapeDtypeStruct((N, 1), jnp.float32),
        scratch_shapes=[
            pltpu.VMEM((N, H), jnp.float32),
            pltpu.VMEM((N, 1), jnp.float32),
            pltpu.VMEM((8, H), jnp.float32),
        ],
    )(h, nb2, epsb, w1, b1, w2, b2, g1, be1, g2, be2, l1w, l1b, l2w, l2b)


def kernel(x, edge_index, c0_eps, c0_w1, c0_b1, c0_w2, c0_b2, c0_g1, c0_be1,
           c0_g2, c0_be2, eps_s, w1_s, b1_s, w2_s, b2_s, g1_s, be1_s, g2_s,
           be2_s, lin1_w, lin1_b, lin2_w, lin2_b):
    x2 = x[:, None]
    x128 = jnp.broadcast_to(x2, (N, HALF))
    src = edge_index[0]
    dst = edge_index[1]

    s2 = _sc_segsum0(x128, src, dst)[:, :N, :1]
    h = _tc_block0(
        x2, s2,
        (1.0 + c0_eps).reshape(1, 1),
        c0_w1.reshape(1, H), c0_b1.reshape(1, H),
        c0_w2, c0_b2.reshape(1, H),
        c0_g1.reshape(1, H), c0_be1.reshape(1, H),
        c0_g2.reshape(1, H), c0_be2.reshape(1, H),
    )

    for i in range(2):
        nb2 = _sc_segsum(h.reshape(2 * N, HALF), src, dst)[:, :N]
        h = _tc_block(
            h, nb2,
            jnp.broadcast_to((1.0 + eps_s[i]).reshape(1, 1), (1, H)),
            w1_s[i], b1_s[i].reshape(1, H), w2_s[i], b2_s[i].reshape(1, H),
            g1_s[i].reshape(1, H), be1_s[i].reshape(1, H),
            g2_s[i].reshape(1, H), be2_s[i].reshape(1, H),
        )

    nb2 = _sc_segsum(h.reshape(2 * N, HALF), src, dst)[:, :N]
    return _tc_final(
        h, nb2,
        jnp.broadcast_to((1.0 + eps_s[2]).reshape(1, 1), (1, H)),
        w1_s[2], b1_s[2].reshape(1, H), w2_s[2], b2_s[2].reshape(1, H),
        g1_s[2].reshape(1, H), be1_s[2].reshape(1, H),
        g2_s[2].reshape(1, H), be2_s[2].reshape(1, H),
        lin1_w, lin1_b.reshape(1, 32), lin2_w, lin2_b.reshape(1, 1),
    )
